# baseline (speedup 1.0000x reference)
import functools

import jax
import jax.numpy as jnp
from jax import lax
from jax.experimental import pallas as pl
from jax.experimental.pallas import tpu as pltpu
from jax.experimental.pallas import tpu_sc as plsc

_B = 8
_N = 5000
_NUM_DET = 3
_IOU_THRESH = 0.5
_L = 16
_TCB = 6
_SCB = _B - _TCB
_SPI = 8
_SH = _N // _SPI
_SHFULL = _SH // _L
_BIG_I32 = 2**31 - 1


def _tc_body(s_ref, bt_ref, cls_ref, out_ref):
    s = s_ref[: _TCB, :]
    x1 = bt_ref[0, : _TCB, :]
    y1 = bt_ref[1, : _TCB, :]
    x2 = bt_ref[2, : _TCB, :]
    y2 = bt_ref[3, : _TCB, :]
    cls = cls_ref[: _TCB, :]
    iota = lax.broadcasted_iota(jnp.int32, s.shape, 1)
    neg_inf = jnp.float32(-jnp.inf)
    zero = jnp.float32(0.0)
    cols = []
    for d in range(_NUM_DET):
        m = jnp.max(s, axis=1, keepdims=True)
        cand = jnp.where(s == m, iota, jnp.int32(_BIG_I32))
        w = jnp.min(cand, axis=1, keepdims=True)
        cols.append(w)
        if d < _NUM_DET - 1:
            sel = iota == w
            wx1 = jnp.sum(jnp.where(sel, x1, zero), axis=1, keepdims=True)
            wy1 = jnp.sum(jnp.where(sel, y1, zero), axis=1, keepdims=True)
            wx2 = jnp.sum(jnp.where(sel, x2, zero), axis=1, keepdims=True)
            wy2 = jnp.sum(jnp.where(sel, y2, zero), axis=1, keepdims=True)
            wcls = jnp.sum(jnp.where(sel, cls, 0), axis=1, keepdims=True)
            ix1 = jnp.maximum(wx1, x1)
            iy1 = jnp.maximum(wy1, y1)
            ix2 = jnp.minimum(wx2, x2)
            iy2 = jnp.minimum(wy2, y2)
            inter = (jnp.maximum(ix2 - ix1, zero) *
                     jnp.maximum(iy2 - iy1, zero))
            warea = (jnp.maximum(wx2 - wx1, zero) *
                     jnp.maximum(wy2 - wy1, zero))
            area_b = (jnp.maximum(x2 - x1, zero) *
                      jnp.maximum(y2 - y1, zero))
            iou = inter / jnp.maximum(warea + area_b - inter,
                                      jnp.float32(1e-9))
            supp = (iou > jnp.float32(_IOU_THRESH)) & (cls == wcls)
            s = jnp.where(supp, neg_inf, s)
    out_ref[...] = jnp.concatenate(cols, axis=1)


def _nms_tc(scores, boxest, classes):
    return pl.pallas_call(
        _tc_body,
        out_shape=jax.ShapeDtypeStruct((_TCB, _NUM_DET), jnp.int32),
    )(scores, boxest, classes)


def _vgather(x, idx):
    dnums = lax.GatherDimensionNumbers(
        offset_dims=(), collapsed_slice_dims=(0,), start_index_map=(0,))
    return lax.gather(x, idx[:, None], dnums, (1,),
                      mode=lax.GatherScatterMode.PROMISE_IN_BOUNDS)


def _butterfly(x, op, lane):
    for sh in (8, 4, 2, 1):
        x = op(x, _vgather(x, lane ^ sh))
    return x


def _sc_body(scores_hbm, boxest_hbm, classes_hbm, out_hbm,
             s_v, x1_v, y1_v, x2_v, y2_v, cls_v, rec_v, grp_v, out_v,
             rows_v, out_2d, sh, sh_rows, sem):
    wid = lax.axis_index("s")
    lane = lax.iota(jnp.int32, _L)
    grp = wid >> 3
    shard = wid & 7
    b = _TCB + grp
    base = shard * _SH

    copies = [
        pltpu.async_copy(scores_hbm.at[b], s_v, sem),
        pltpu.async_copy(boxest_hbm.at[0, b], x1_v, sem),
        pltpu.async_copy(boxest_hbm.at[1, b], y1_v, sem),
        pltpu.async_copy(boxest_hbm.at[2, b], x2_v, sem),
        pltpu.async_copy(boxest_hbm.at[3, b], y2_v, sem),
        pltpu.async_copy(classes_hbm.at[b], cls_v, sem),
    ]
    for cp in copies:
        cp.wait()

    neg_inf = jnp.float32(-jnp.inf)
    bv0 = jnp.full((_L,), neg_inf, jnp.float32)
    bi0 = jnp.zeros((_L,), jnp.int32)

    def pass_a(rel, carry):
        bv, bi = carry
        off = base + rel
        sv = s_v[pl.ds(off, _L)]
        idx = off + lane
        cond = sv > bv
        return jnp.where(cond, sv, bv), jnp.where(cond, idx, bi)

    def publish(carry):
        bv, bi = carry
        m = _butterfly(bv, jnp.maximum, lane)
        cand = jnp.where(bv == m, bi, jnp.int32(_BIG_I32))
        wi = _butterfly(cand, jnp.minimum, lane)
        rec = jnp.where(lane == 0, m,
                        plsc.bitcast(wi, jnp.float32))
        rec_v[...] = rec
        pltpu.sync_copy(rec_v, sh.at[pl.ds(wid * _L, _L)])

    def combine():
        pltpu.sync_copy(sh.at[pl.ds(grp * (_SPI * _L), _SPI * _L)], grp_v)
        ri = jnp.minimum(lane, _SPI - 1) * _L
        vals = plsc.load_gather(grp_v, [ri])
        idxs = plsc.bitcast(plsc.load_gather(grp_v, [ri + 1]), jnp.int32)
        m = _butterfly(vals, jnp.maximum, lane)
        cand = jnp.where(vals == m, idxs, jnp.int32(_BIG_I32))
        wv = _butterfly(cand, jnp.minimum, lane)
        wx1 = plsc.load_gather(x1_v, [wv])
        wy1 = plsc.load_gather(y1_v, [wv])
        wx2 = plsc.load_gather(x2_v, [wv])
        wy2 = plsc.load_gather(y2_v, [wv])
        wcls = plsc.load_gather(cls_v, [wv])
        warea = (jnp.maximum(wx2 - wx1, jnp.float32(0.0)) *
                 jnp.maximum(wy2 - wy1, jnp.float32(0.0)))
        return wv, (wx1, wy1, wx2, wy2, wcls, warea)

    def fused_body(wd, rel, carry):
        wx1, wy1, wx2, wy2, wcls, warea = wd
        bv, bi = carry
        off = base + rel
        sl = pl.ds(off, _L)
        x1c = x1_v[sl]
        y1c = y1_v[sl]
        x2c = x2_v[sl]
        y2c = y2_v[sl]
        ix1 = jnp.maximum(wx1, x1c)
        iy1 = jnp.maximum(wy1, y1c)
        ix2 = jnp.minimum(wx2, x2c)
        iy2 = jnp.minimum(wy2, y2c)
        inter = (jnp.maximum(ix2 - ix1, jnp.float32(0.0)) *
                 jnp.maximum(iy2 - iy1, jnp.float32(0.0)))
        area_b = (jnp.maximum(x2c - x1c, jnp.float32(0.0)) *
                  jnp.maximum(y2c - y1c, jnp.float32(0.0)))
        iou = inter / jnp.maximum(warea + area_b - inter, jnp.float32(1e-9))
        supp = (iou > jnp.float32(_IOU_THRESH)) & (cls_v[sl] == wcls)
        sv = jnp.where(supp, neg_inf, s_v[sl])
        s_v[sl] = sv
        idx = off + lane
        cond = sv > bv
        return jnp.where(cond, sv, bv), jnp.where(cond, idx, bi)

    def sweep(body, carry):
        carry = plsc.parallel_loop(
            0, _SHFULL * _L, _L, unroll=4, carry=carry)(body)
        return body(_SH - _L, carry)

    out_v[...] = jnp.zeros((_L,), jnp.int32)
    carry = sweep(pass_a, (bv0, bi0))
    for d in range(_NUM_DET):
        publish(carry)
        plsc.subcore_barrier()
        wv, wd = combine()
        plsc.subcore_barrier()
        out_v[...] = jnp.where(lane == d, wv, out_v[...])
        if d < _NUM_DET - 1:
            carry = sweep(functools.partial(fused_body, wd), (bv0, bi0))

    @pl.when(shard == 0)
    def _():
        pltpu.sync_copy(out_v, sh_rows.at[pl.ds(grp * _L, _L)])
    plsc.subcore_barrier()

    @pl.when(wid == 0)
    def _():
        pltpu.sync_copy(sh_rows, rows_v)
        k = lane
        q2 = (k * 21846) >> 16
        r = k - q2 * 3
        src = jnp.minimum(q2 * _L + r, _SCB * _L - 1)
        vals = plsc.load_gather(rows_v, [src])
        plsc.store_scatter(out_2d, [jnp.minimum(q2, _SCB - 1), r], vals,
                           mask=k < _SCB * _NUM_DET)
        pltpu.sync_copy(out_2d, out_hbm)


def _nms_sc(scores, boxest, classes):
    mesh = plsc.VectorSubcoreMesh(core_axis_name="c", subcore_axis_name="s",
                                  num_cores=1)
    f = pl.kernel(
        _sc_body,
        out_type=jax.ShapeDtypeStruct((_SCB, _NUM_DET), jnp.int32),
        mesh=mesh,
        scratch_types=[
            pltpu.VMEM((_N,), jnp.float32),
            pltpu.VMEM((_N,), jnp.float32),
            pltpu.VMEM((_N,), jnp.float32),
            pltpu.VMEM((_N,), jnp.float32),
            pltpu.VMEM((_N,), jnp.float32),
            pltpu.VMEM((_N,), jnp.int32),
            pltpu.VMEM((_L,), jnp.float32),
            pltpu.VMEM((_SPI * _L,), jnp.float32),
            pltpu.VMEM((_L,), jnp.int32),
            pltpu.VMEM((_SCB * _L,), jnp.int32),
            pltpu.VMEM((_SCB, _NUM_DET), jnp.int32),
            pltpu.VMEM_SHARED((16 * _L,), jnp.float32),
            pltpu.VMEM_SHARED((_SCB * _L,), jnp.int32),
            pltpu.SemaphoreType.DMA,
        ],
        compiler_params=pltpu.CompilerParams(needs_layout_passes=False),
    )
    return f(scores, boxest, classes)


@jax.jit
def _nms(scores, boxest, classes):
    tc_out = _nms_tc(scores, boxest, classes)
    sc_out = _nms_sc(scores, boxest, classes)
    return jnp.concatenate([tc_out, sc_out], axis=0)


def kernel(scores, boxes, classes):
    return _nms(scores, boxes.transpose(2, 0, 1), classes)


# speedup vs baseline: 1.0800x; 1.0800x over previous
import functools

import jax
import jax.numpy as jnp
from jax import lax
from jax.experimental import pallas as pl
from jax.experimental.pallas import tpu as pltpu
from jax.experimental.pallas import tpu_sc as plsc

_B = 8
_N = 5000
_NP = 5120
_NUM_DET = 3
_IOU_THRESH = 0.5
_L = 16
_TCB = 6
_SCB = _B - _TCB
_SPI = 8
_SH = _NP // _SPI
_SHCHUNK = _SH // _L
_BIG_I32 = 2**31 - 1


def _tc_body(s_ref, bt_ref, cls_ref, out_ref):
    s = s_ref[: _TCB, :]
    x1 = bt_ref[: _TCB, 0, :]
    y1 = bt_ref[: _TCB, 1, :]
    x2 = bt_ref[: _TCB, 2, :]
    y2 = bt_ref[: _TCB, 3, :]
    cls = cls_ref[: _TCB, :]
    iota = lax.broadcasted_iota(jnp.int32, s.shape, 1)
    neg_inf = jnp.float32(-jnp.inf)
    zero = jnp.float32(0.0)
    cols = []
    for d in range(_NUM_DET):
        m = jnp.max(s, axis=1, keepdims=True)
        cand = jnp.where(s == m, iota, jnp.int32(_BIG_I32))
        w = jnp.min(cand, axis=1, keepdims=True)
        cols.append(w)
        if d < _NUM_DET - 1:
            sel = iota == w
            wx1 = jnp.sum(jnp.where(sel, x1, zero), axis=1, keepdims=True)
            wy1 = jnp.sum(jnp.where(sel, y1, zero), axis=1, keepdims=True)
            wx2 = jnp.sum(jnp.where(sel, x2, zero), axis=1, keepdims=True)
            wy2 = jnp.sum(jnp.where(sel, y2, zero), axis=1, keepdims=True)
            wcls = jnp.sum(jnp.where(sel, cls, 0), axis=1, keepdims=True)
            ix1 = jnp.maximum(wx1, x1)
            iy1 = jnp.maximum(wy1, y1)
            ix2 = jnp.minimum(wx2, x2)
            iy2 = jnp.minimum(wy2, y2)
            inter = (jnp.maximum(ix2 - ix1, zero) *
                     jnp.maximum(iy2 - iy1, zero))
            warea = (jnp.maximum(wx2 - wx1, zero) *
                     jnp.maximum(wy2 - wy1, zero))
            area_b = (jnp.maximum(x2 - x1, zero) *
                      jnp.maximum(y2 - y1, zero))
            iou = inter / jnp.maximum(warea + area_b - inter,
                                      jnp.float32(1e-9))
            supp = (iou > jnp.float32(_IOU_THRESH)) & (cls == wcls)
            s = jnp.where(supp, neg_inf, s)
    out_ref[...] = jnp.concatenate(cols, axis=1)


def _nms_tc(scores_p, boxest_p, classes_p):
    return pl.pallas_call(
        _tc_body,
        out_shape=jax.ShapeDtypeStruct((_TCB, _NUM_DET), jnp.int32),
    )(scores_p, boxest_p, classes_p)


def _vgather(x, idx):
    dnums = lax.GatherDimensionNumbers(
        offset_dims=(), collapsed_slice_dims=(0,), start_index_map=(0,))
    return lax.gather(x, idx[:, None], dnums, (1,),
                      mode=lax.GatherScatterMode.PROMISE_IN_BOUNDS)


def _butterfly(x, op, lane):
    for sh in (8, 4, 2, 1):
        x = op(x, _vgather(x, lane ^ sh))
    return x


def _sc_body(scores_hbm, boxest_hbm, classes_hbm, out_hbm,
             s_v, x1_v, y1_v, x2_v, y2_v, cls_v, rec_v, grp_v, out_v,
             rows_v, out_2d, sh, sh_rows, sem):
    wid = lax.axis_index("s")
    lane = lax.iota(jnp.int32, _L)
    grp = wid >> 3
    q = wid & 7
    b = _TCB + grp
    base = q * _SH

    copies = [
        pltpu.async_copy(scores_hbm.at[b, pl.ds(base, _SH)], s_v, sem),
        pltpu.async_copy(boxest_hbm.at[b, 0, pl.ds(base, _SH)], x1_v, sem),
        pltpu.async_copy(boxest_hbm.at[b, 1, pl.ds(base, _SH)], y1_v, sem),
        pltpu.async_copy(boxest_hbm.at[b, 2, pl.ds(base, _SH)], x2_v, sem),
        pltpu.async_copy(boxest_hbm.at[b, 3, pl.ds(base, _SH)], y2_v, sem),
        pltpu.async_copy(classes_hbm.at[b, pl.ds(base, _SH)], cls_v, sem),
    ]
    for cp in copies:
        cp.wait()

    neg_inf = jnp.float32(-jnp.inf)
    bv0 = jnp.full((_L,), neg_inf, jnp.float32)
    bi0 = jnp.zeros((_L,), jnp.int32)

    def pass_a(off, carry):
        bv, bi = carry
        sv = s_v[pl.ds(off, _L)]
        idx = base + off + lane
        cond = sv > bv
        return jnp.where(cond, sv, bv), jnp.where(cond, idx, bi)

    def publish(carry):
        bv, bi = carry
        m = _butterfly(bv, jnp.maximum, lane)
        cand = jnp.where(bv == m, bi, jnp.int32(_BIG_I32))
        wi = _butterfly(cand, jnp.minimum, lane)
        p = jnp.maximum(wi - base, 0)
        wx1 = plsc.load_gather(x1_v, [p])
        wy1 = plsc.load_gather(y1_v, [p])
        wx2 = plsc.load_gather(x2_v, [p])
        wy2 = plsc.load_gather(y2_v, [p])
        wclsf = plsc.bitcast(plsc.load_gather(cls_v, [p]), jnp.float32)
        wif = plsc.bitcast(wi, jnp.float32)
        rec = jnp.where(lane == 0, m,
              jnp.where(lane == 1, wif,
              jnp.where(lane == 2, wx1,
              jnp.where(lane == 3, wy1,
              jnp.where(lane == 4, wx2,
              jnp.where(lane == 5, wy2,
              jnp.where(lane == 6, wclsf, jnp.float32(0.0))))))))
        rec_v[...] = rec
        pltpu.sync_copy(rec_v, sh.at[pl.ds(wid * _L, _L)])

    def combine():
        pltpu.sync_copy(sh.at[pl.ds(grp * (_SPI * _L), _SPI * _L)], grp_v)
        ri = jnp.minimum(lane, _SPI - 1) * _L
        vals = plsc.load_gather(grp_v, [ri])
        idxs = plsc.bitcast(plsc.load_gather(grp_v, [ri + 1]), jnp.int32)
        m = _butterfly(vals, jnp.maximum, lane)
        cand = jnp.where(vals == m, idxs, jnp.int32(_BIG_I32))
        wv = _butterfly(cand, jnp.minimum, lane)
        rsel = jnp.where((vals == m) & (idxs == wv),
                         jnp.minimum(lane, _SPI - 1), jnp.int32(_BIG_I32))
        rb = _butterfly(rsel, jnp.minimum, lane) * _L
        wx1 = plsc.load_gather(grp_v, [rb + 2])
        wy1 = plsc.load_gather(grp_v, [rb + 3])
        wx2 = plsc.load_gather(grp_v, [rb + 4])
        wy2 = plsc.load_gather(grp_v, [rb + 5])
        wcls = plsc.bitcast(plsc.load_gather(grp_v, [rb + 6]), jnp.int32)
        warea = (jnp.maximum(wx2 - wx1, jnp.float32(0.0)) *
                 jnp.maximum(wy2 - wy1, jnp.float32(0.0)))
        return wv, (wx1, wy1, wx2, wy2, wcls, warea)

    def fused_body(wd, off, carry):
        wx1, wy1, wx2, wy2, wcls, warea = wd
        bv, bi = carry
        sl = pl.ds(off, _L)
        x1c = x1_v[sl]
        y1c = y1_v[sl]
        x2c = x2_v[sl]
        y2c = y2_v[sl]
        ix1 = jnp.maximum(wx1, x1c)
        iy1 = jnp.maximum(wy1, y1c)
        ix2 = jnp.minimum(wx2, x2c)
        iy2 = jnp.minimum(wy2, y2c)
        inter = (jnp.maximum(ix2 - ix1, jnp.float32(0.0)) *
                 jnp.maximum(iy2 - iy1, jnp.float32(0.0)))
        area_b = (jnp.maximum(x2c - x1c, jnp.float32(0.0)) *
                  jnp.maximum(y2c - y1c, jnp.float32(0.0)))
        iou = inter / jnp.maximum(warea + area_b - inter, jnp.float32(1e-9))
        supp = (iou > jnp.float32(_IOU_THRESH)) & (cls_v[sl] == wcls)
        sv = jnp.where(supp, neg_inf, s_v[sl])
        s_v[sl] = sv
        idx = base + off + lane
        cond = sv > bv
        return jnp.where(cond, sv, bv), jnp.where(cond, idx, bi)

    def sweep(body, carry):
        return plsc.parallel_loop(
            0, _SH, _L, unroll=4, carry=carry)(body)

    out_v[...] = jnp.zeros((_L,), jnp.int32)
    carry = sweep(pass_a, (bv0, bi0))
    for d in range(_NUM_DET):
        publish(carry)
        plsc.subcore_barrier()
        wv, wd = combine()
        plsc.subcore_barrier()
        out_v[...] = jnp.where(lane == d, wv, out_v[...])
        if d < _NUM_DET - 1:
            carry = sweep(functools.partial(fused_body, wd), (bv0, bi0))

    @pl.when(q == 0)
    def _():
        pltpu.sync_copy(out_v, sh_rows.at[pl.ds(grp * _L, _L)])
    plsc.subcore_barrier()

    @pl.when(wid == 0)
    def _():
        pltpu.sync_copy(sh_rows, rows_v)
        k = lane
        q2 = (k * 21846) >> 16
        r = k - q2 * 3
        src = jnp.minimum(q2 * _L + r, _SCB * _L - 1)
        vals = plsc.load_gather(rows_v, [src])
        plsc.store_scatter(out_2d, [jnp.minimum(q2, _SCB - 1), r], vals,
                           mask=k < _SCB * _NUM_DET)
        pltpu.sync_copy(out_2d, out_hbm)


def _nms_sc(scores_p, boxest_p, classes_p):
    mesh = plsc.VectorSubcoreMesh(core_axis_name="c", subcore_axis_name="s",
                                  num_cores=1)
    f = pl.kernel(
        _sc_body,
        out_type=jax.ShapeDtypeStruct((_SCB, _NUM_DET), jnp.int32),
        mesh=mesh,
        scratch_types=[
            pltpu.VMEM((_SH,), jnp.float32),
            pltpu.VMEM((_SH,), jnp.float32),
            pltpu.VMEM((_SH,), jnp.float32),
            pltpu.VMEM((_SH,), jnp.float32),
            pltpu.VMEM((_SH,), jnp.float32),
            pltpu.VMEM((_SH,), jnp.int32),
            pltpu.VMEM((_L,), jnp.float32),
            pltpu.VMEM((_SPI * _L,), jnp.float32),
            pltpu.VMEM((_L,), jnp.int32),
            pltpu.VMEM((_SCB * _L,), jnp.int32),
            pltpu.VMEM((_SCB, _NUM_DET), jnp.int32),
            pltpu.VMEM_SHARED((16 * _L,), jnp.float32),
            pltpu.VMEM_SHARED((_SCB * _L,), jnp.int32),
            pltpu.SemaphoreType.DMA,
        ],
        compiler_params=pltpu.CompilerParams(needs_layout_passes=False),
    )
    return f(scores_p, boxest_p, classes_p)


@jax.jit
def _nms(scores, boxest, classes):
    pad = _NP - _N
    scores_p = jnp.pad(scores, ((0, 0), (0, pad)), constant_values=-jnp.inf)
    boxest_p = jnp.pad(boxest, ((0, 0), (0, 0), (0, pad)))
    classes_p = jnp.pad(classes, ((0, 0), (0, pad)))
    tc_out = _nms_tc(scores_p, boxest_p, classes_p)
    sc_out = _nms_sc(scores_p, boxest_p, classes_p)
    return jnp.concatenate([tc_out, sc_out], axis=0)


def kernel(scores, boxes, classes):
    return _nms(scores, boxes.transpose(0, 2, 1), classes)
